# baseline (speedup 1.0000x reference)
import jax
import jax.numpy as jnp
from jax.experimental import pallas as pl
from jax.experimental.pallas import tpu as pltpu

_B, _C, _H, _W = 8, 384, 200, 176
_O_PAD = 32
_G = 8
_NG = _B * (_H // _G)


def _head_kernel(x_hbm, w_ref, b_ref, cls_ref, reg_ref, dir_ref, xbuf, sems):
    bi = pl.program_id(0)
    hi = pl.program_id(1)
    flat = bi * (_H // _G) + hi

    def start_group(g, slot):
        b2 = g // (_H // _G)
        h0 = (g % (_H // _G)) * _G
        for j in range(_G):
            pltpu.make_async_copy(
                x_hbm.at[b2, :, h0 + j, :], xbuf.at[slot, j], sems.at[slot, j]
            ).start()

    @pl.when(flat == 0)
    def _():
        start_group(flat, flat % 2)

    @pl.when(flat + 1 < _NG)
    def _():
        start_group(flat + 1, (flat + 1) % 2)

    slot = flat % 2
    w = w_ref[...]
    b = b_ref[...]
    for j in range(_G):
        pltpu.make_async_copy(
            x_hbm.at[0, :, 0, :], xbuf.at[slot, j], sems.at[slot, j]
        ).wait()
        acc = jax.lax.dot_general(
            w, xbuf[slot, j],
            dimension_numbers=(((1,), (0,)), ((), ())),
            preferred_element_type=jnp.float32,
        ) + b
        cls_ref[0, :, j, :] = acc[0:2]
        reg_ref[0, :, j, :] = acc[2:16]
        dir_ref[0, :, j, :] = acc[16:20]


def kernel(x, W_cls, b_cls, W_reg, b_reg, W_dir, b_dir):
    w = jnp.concatenate([W_cls, W_reg, W_dir], axis=1).T
    w = jnp.pad(w, ((0, _O_PAD - w.shape[0]), (0, 0)))
    b = jnp.concatenate([b_cls, b_reg, b_dir])
    b = jnp.pad(b, (0, _O_PAD - b.shape[0]))[:, None]

    cls_o, reg_o, dir_o = pl.pallas_call(
        _head_kernel,
        grid=(_B, _H // _G),
        in_specs=[
            pl.BlockSpec(memory_space=pltpu.MemorySpace.HBM),
            pl.BlockSpec((_O_PAD, _C), lambda bi, hi: (0, 0)),
            pl.BlockSpec((_O_PAD, 1), lambda bi, hi: (0, 0)),
        ],
        out_specs=[
            pl.BlockSpec((1, 2, _G, _W), lambda bi, hi: (bi, 0, hi, 0)),
            pl.BlockSpec((1, 14, _G, _W), lambda bi, hi: (bi, 0, hi, 0)),
            pl.BlockSpec((1, 4, _G, _W), lambda bi, hi: (bi, 0, hi, 0)),
        ],
        out_shape=[
            jax.ShapeDtypeStruct((_B, 2, _H, _W), jnp.float32),
            jax.ShapeDtypeStruct((_B, 14, _H, _W), jnp.float32),
            jax.ShapeDtypeStruct((_B, 4, _H, _W), jnp.float32),
        ],
        scratch_shapes=[
            pltpu.VMEM((2, _G, _C, _W), jnp.float32),
            pltpu.SemaphoreType.DMA((2, _G)),
        ],
        compiler_params=pltpu.CompilerParams(
            dimension_semantics=("arbitrary", "arbitrary"),
        ),
    )(x, w, b)

    return (cls_o, reg_o, dir_o)


# speedup vs baseline: 1.2978x; 1.2978x over previous
import jax
import jax.numpy as jnp
from jax.experimental import pallas as pl
from jax.experimental.pallas import tpu as pltpu

_B, _C, _H, _W = 8, 384, 200, 176
_HW = _H * _W
_O_PAD = 32
_NW = 7040


def _head_kernel(x_ref, w_ref, b_ref, o_ref):
    o_ref[0] = jax.lax.dot_general(
        w_ref[...], x_ref[0],
        dimension_numbers=(((1,), (0,)), ((), ())),
        preferred_element_type=jnp.float32,
    ) + b_ref[...]


def kernel(x, W_cls, b_cls, W_reg, b_reg, W_dir, b_dir):
    w = jnp.concatenate([W_cls, W_reg, W_dir], axis=1).T
    w = jnp.pad(w, ((0, _O_PAD - w.shape[0]), (0, 0)))
    b = jnp.concatenate([b_cls, b_reg, b_dir])
    b = jnp.pad(b, (0, _O_PAD - b.shape[0]))[:, None]

    x3 = x.reshape(_B, _C, _HW)
    n_blocks = _HW // _NW

    out = pl.pallas_call(
        _head_kernel,
        grid=(_B, n_blocks),
        in_specs=[
            pl.BlockSpec((1, _C, _NW), lambda bi, ni: (bi, 0, ni)),
            pl.BlockSpec((_O_PAD, _C), lambda bi, ni: (0, 0)),
            pl.BlockSpec((_O_PAD, 1), lambda bi, ni: (0, 0)),
        ],
        out_specs=pl.BlockSpec((1, _O_PAD, _NW), lambda bi, ni: (bi, 0, ni)),
        out_shape=jax.ShapeDtypeStruct((_B, _O_PAD, _HW), jnp.float32),
        compiler_params=pltpu.CompilerParams(
            dimension_semantics=("parallel", "parallel"),
        ),
    )(x3, w, b)

    out4 = out.reshape(_B, _O_PAD, _H, _W)
    return (out4[:, 0:2], out4[:, 2:16], out4[:, 16:20])
